# baseline (speedup 1.0000x reference)
import jax
import jax.numpy as jnp
from jax.experimental import pallas as pl
from jax.experimental.pallas import tpu as pltpu
from jax.experimental.pallas import tpu_sc as plsc

_WINDOW = 128


def kernel(x, embedding_weight):
    B, S = x.shape
    V, D = embedding_weight.shape
    n = B * S
    idx = x.reshape(1, n).astype(jnp.int32)

    mesh = plsc.VectorSubcoreMesh(
        core_axis_name="core", subcore_axis_name="subcore"
    )

    @pl.kernel(
        out_type=jax.ShapeDtypeStruct((n, D), embedding_weight.dtype),
        mesh=mesh,
    )
    def gather_kernel(table_hbm, idx_hbm, out_hbm):
        def body(i_vmem, o_vmem):
            pltpu.sync_copy(table_hbm.at[i_vmem.at[0]], o_vmem)

        pltpu.emit_pipeline(
            body,
            grid=(n // _WINDOW,),
            in_specs=[pl.BlockSpec((1, _WINDOW), index_map=lambda i: (0, i))],
            out_specs=[pl.BlockSpec((_WINDOW, D), index_map=lambda i: (i, 0))],
            core_axis_name=("core", "subcore"),
            dimension_semantics=(pltpu.PARALLEL,),
        )(idx_hbm, out_hbm)

    out = gather_kernel(embedding_weight, idx)
    return out.reshape(B, S, D)


# speedup vs baseline: 1.2372x; 1.2372x over previous
import jax
import jax.numpy as jnp
from jax.experimental import pallas as pl
from jax.experimental.pallas import tpu as pltpu
from jax.experimental.pallas import tpu_sc as plsc

_WINDOW = 256


def kernel(x, embedding_weight):
    B, S = x.shape
    V, D = embedding_weight.shape
    n = B * S
    idx = x.reshape(1, n).astype(jnp.int32)

    mesh = plsc.VectorSubcoreMesh(
        core_axis_name="core", subcore_axis_name="subcore"
    )

    @pl.kernel(
        out_type=jax.ShapeDtypeStruct((n, D), embedding_weight.dtype),
        mesh=mesh,
    )
    def gather_kernel(table_hbm, idx_hbm, out_hbm):
        def body(i_vmem, o_vmem):
            pltpu.sync_copy(table_hbm.at[i_vmem.at[0]], o_vmem)

        pltpu.emit_pipeline(
            body,
            grid=(n // _WINDOW,),
            in_specs=[pl.BlockSpec((1, _WINDOW), index_map=lambda i: (0, i))],
            out_specs=[pl.BlockSpec((_WINDOW, D), index_map=lambda i: (i, 0))],
            core_axis_name=("core", "subcore"),
            dimension_semantics=(pltpu.PARALLEL,),
        )(idx_hbm, out_hbm)

    out = gather_kernel(embedding_weight, idx)
    return out.reshape(B, S, D)


# speedup vs baseline: 1.2507x; 1.0109x over previous
import jax
from jax import lax
import jax.numpy as jnp
from jax.experimental import pallas as pl
from jax.experimental.pallas import tpu as pltpu
from jax.experimental.pallas import tpu_sc as plsc

_W = 256
_NC = 2
_NS = 16
_NT = _NC * _NS


def kernel(x, embedding_weight):
    B, S = x.shape
    V, D = embedding_weight.shape
    n = B * S
    per_tile = n // _NT
    nsteps = per_tile // _W
    idx = x.reshape(n).astype(jnp.int32)

    mesh = plsc.VectorSubcoreMesh(
        core_axis_name="core", subcore_axis_name="subcore"
    )

    @pl.kernel(
        out_type=jax.ShapeDtypeStruct((n, D), embedding_weight.dtype),
        mesh=mesh,
        scratch_types=[
            pltpu.VMEM((per_tile,), jnp.int32),
            pltpu.VMEM((_W, D), jnp.float32),
            pltpu.VMEM((_W, D), jnp.float32),
            pltpu.SemaphoreType.DMA,
            pltpu.SemaphoreType.DMA,
            pltpu.SemaphoreType.DMA,
        ],
    )
    def gather_kernel(table_hbm, idx_hbm, out_hbm, idx_v, buf0, buf1,
                      gsem, wsem0, wsem1):
        tile = lax.axis_index("subcore") * _NC + lax.axis_index("core")
        base = tile * per_tile
        pltpu.sync_copy(idx_hbm.at[pl.ds(base, per_tile)], idx_v)

        bufs = (buf0, buf1)
        wsems = (wsem0, wsem1)

        @pl.loop(0, nsteps, step=2)
        def _(s):
            for b in range(2):
                st = s + b
                off = base + st * _W

                @pl.when(st >= 2)
                def _():
                    pltpu.make_async_copy(
                        bufs[b], out_hbm.at[pl.ds(off, _W)], wsems[b]
                    ).wait()

                pltpu.async_copy(
                    table_hbm.at[idx_v.at[pl.ds(st * _W, _W)]],
                    bufs[b],
                    gsem,
                ).wait()
                pltpu.async_copy(
                    bufs[b], out_hbm.at[pl.ds(off, _W)], wsems[b]
                )

        for b in range(2):
            st = nsteps - 2 + b
            pltpu.make_async_copy(
                bufs[b], out_hbm.at[pl.ds(base + st * _W, _W)], wsems[b]
            ).wait()

    out = gather_kernel(embedding_weight, idx)
    return out.reshape(B, S, D)
